# baseline (speedup 1.0000x reference)
import functools

import jax
import jax.numpy as jnp
from jax import lax
from jax.experimental import pallas as pl
from jax.experimental.pallas import tpu as pltpu
from jax.experimental.pallas import tpu_sc as plsc

N = 16384
D = 64
DH = D // 2
NT = 16
CHUNK = 512
IDXW = 128
NSUB = CHUNK // IDXW
SUP = 8
SROWS = SUP * CHUNK // IDXW
SCH = SUP * CHUNK


def _sc_body(nsup, inp_hbm, si_hbm, sv_hbm, tail_hbm, dflat_hbm, out_hbm,
             cbuf, vbuf, r2d, pbuf, gat, acc, sem_p, sem_g, sem_s):
    c = lax.axis_index("c")
    s = lax.axis_index("s")
    coff = c * N
    tile_base = s * (nsup * SCH)

    def _zb(i, _):
        gat[0, i, pl.ds(0, 16)] = jnp.zeros((16,), jnp.float32)
        gat[0, i, pl.ds(16, 16)] = jnp.zeros((16,), jnp.float32)
        return _
    lax.fori_loop(0, CHUNK, _zb, None)
    arows = N // NT
    pltpu.sync_copy(gat.at[0], acc.at[pl.ds(s * arows, CHUNK)])
    pltpu.sync_copy(gat.at[0], acc.at[pl.ds(s * arows + CHUNK, CHUNK)])
    plsc.subcore_barrier()

    def meta_copies(u, b):
        base = tile_base + u * SCH
        cps = [pltpu.make_async_copy(si_hbm.at[1, pl.ds(base, SCH)],
                                     cbuf.at[b], sem_p.at[b]),
               pltpu.make_async_copy(sv_hbm.at[pl.ds(base, SCH)],
                                     vbuf.at[b], sem_p.at[b])]
        cps += [pltpu.make_async_copy(si_hbm.at[0, pl.ds(base + r * IDXW, IDXW)],
                                      r2d.at[b, r], sem_p.at[b])
                for r in range(SROWS)]
        return cps

    def fire_meta(u, b):
        for cp in meta_copies(u, b):
            cp.start()

    def wait_meta(u, b):
        for cp in meta_copies(u, b):
            cp.wait()

    def _off(b):
        def body(i, _):
            cbuf[b, pl.ds(i * 16, 16)] = cbuf[b, pl.ds(i * 16, 16)] + coff
            return _
        lax.fori_loop(0, SCH // 16, body, None)

    def _sup(b, cols_at, rows_at, vals_vec):
        pend_g, pend_s = {}, {}

        def fire_gather(k):
            g = k % 2
            pend_g[k] = [
                pltpu.async_copy(dflat_hbm.at[cols_at(k, j)],
                                 gat.at[g, pl.ds(j * IDXW, IDXW)], sem_g.at[g])
                for j in range(NSUB)]

        def scale_scatter(k):
            g = k % 2
            for cp in pend_g.pop(k):
                cp.wait()

            def _mul(i, _):
                vv = vals_vec(k * (CHUNK // 16) + i)
                for t in range(16):
                    q = i * 16 + t
                    gat[g, q, pl.ds(0, 16)] = gat[g, q, pl.ds(0, 16)] * vv[t]
                    gat[g, q, pl.ds(16, 16)] = gat[g, q, pl.ds(16, 16)] * vv[t]
                return _
            lax.fori_loop(0, CHUNK // 16, _mul, None)

            pend_s[k] = [
                pltpu.async_copy(gat.at[g, pl.ds(j * IDXW, IDXW)],
                                 acc.at[rows_at(k, j)], sem_s.at[g], add=True)
                for j in range(NSUB)]

        for k in range(SUP):
            if k >= 2:
                for cp in pend_s.pop(k - 2):
                    cp.wait()
            fire_gather(k)
            if k >= 1:
                scale_scatter(k - 1)
        scale_scatter(SUP - 1)
        for kk in (SUP - 2, SUP - 1):
            for cp in pend_s.pop(kk):
                cp.wait()

    def _sup_raw(u, b):
        wait_meta(u, b)
        un = jnp.minimum(u + 1, nsup - 1)
        fire_meta(un, 1 - b)
        _off(b)
        _sup(b,
             lambda k, j: cbuf.at[b, pl.ds(k * CHUNK + j * IDXW, IDXW)],
             lambda k, j: r2d.at[b, k * NSUB + j],
             lambda i: vbuf[b, pl.ds(i * 16, 16)])

    fire_meta(0, 0)

    def _pair(u2, _):
        _sup_raw(2 * u2, 0)
        _sup_raw(2 * u2 + 1, 1)
        return _
    lax.fori_loop(0, nsup // 2, _pair, None)
    if nsup % 2:
        _sup_raw(nsup - 1, 0)
    wait_meta(nsup - 1, nsup % 2)

    pltpu.sync_copy(tail_hbm.at[pl.ds(s * SROWS, SROWS)], pbuf)

    def _offt(i, _):
        r, l = i // 8, (i % 8) * 16
        pbuf[r, 0, pl.ds(l, 16)] = pbuf[r, 0, pl.ds(l, 16)] + coff
        return _
    lax.fori_loop(0, SCH // 16, _offt, None)
    _sup(0,
         lambda k, j: pbuf.at[k * NSUB + j, 0],
         lambda k, j: pbuf.at[k * NSUB + j, 1],
         lambda i: plsc.bitcast(pbuf[i // 8, 2, pl.ds((i % 8) * 16, 16)],
                                jnp.float32))

    plsc.subcore_barrier()

    for half in range(2):
        r0 = s * arows + half * CHUNK
        pltpu.sync_copy(inp_hbm.at[pl.ds(r0, CHUNK), pl.ds(c * DH, DH)],
                        gat.at[0])
        pltpu.sync_copy(acc.at[pl.ds(r0, CHUNK)], gat.at[1])

        def _add(i, _):
            gat[0, i, pl.ds(0, 16)] = gat[0, i, pl.ds(0, 16)] + gat[1, i, pl.ds(0, 16)]
            gat[0, i, pl.ds(16, 16)] = gat[0, i, pl.ds(16, 16)] + gat[1, i, pl.ds(16, 16)]
            return _
        lax.fori_loop(0, CHUNK, _add, None)
        pltpu.sync_copy(gat.at[0],
                        out_hbm.at[pl.ds(r0, CHUNK), pl.ds(c * DH, DH)])


def kernel(input_mat, sparse_indices, sparse_values, dense):
    nnz = sparse_values.shape[0]
    quantum = NT * SCH
    nsup = nnz // quantum
    bulk = nsup * quantum
    tail_n = nnz - bulk
    pad = quantum - tail_n
    ar = jnp.arange(pad, dtype=jnp.int32)
    rows_t = jnp.concatenate([sparse_indices[0, bulk:], (ar * 97) % N])
    cols_t = jnp.concatenate([sparse_indices[1, bulk:], (ar * 89) % N])
    vals_t = jnp.pad(sparse_values[bulk:], (0, pad))
    tail = jnp.stack([
        cols_t.reshape(-1, IDXW),
        rows_t.reshape(-1, IDXW),
        lax.bitcast_convert_type(vals_t, jnp.int32).reshape(-1, IDXW),
    ], axis=1)
    dflat = jnp.concatenate([dense[:, :DH], dense[:, DH:]], axis=0)

    mesh = plsc.VectorSubcoreMesh(core_axis_name="c", subcore_axis_name="s")
    body = functools.partial(_sc_body, nsup)
    return pl.kernel(
        body,
        out_type=jax.ShapeDtypeStruct((N, D), jnp.float32),
        mesh=mesh,
        compiler_params=pltpu.CompilerParams(use_tc_tiling_on_sc=False,
                                             needs_layout_passes=False),
        scratch_types=[
            pltpu.VMEM((2, SCH), jnp.int32),
            pltpu.VMEM((2, SCH), jnp.float32),
            pltpu.VMEM((2, SROWS, IDXW), jnp.int32),
            pltpu.VMEM((SROWS, 3, IDXW), jnp.int32),
            pltpu.VMEM((2, CHUNK, DH), jnp.float32),
            pltpu.VMEM_SHARED((N, DH), jnp.float32),
            pltpu.SemaphoreType.DMA((2,)),
            pltpu.SemaphoreType.DMA((2,)),
            pltpu.SemaphoreType.DMA((2,)),
        ],
    )(input_mat, sparse_indices, sparse_values, tail, dflat)


# speedup vs baseline: 1.3601x; 1.3601x over previous
import functools

import jax
import jax.numpy as jnp
from jax import lax
from jax.experimental import pallas as pl
from jax.experimental.pallas import tpu as pltpu
from jax.experimental.pallas import tpu_sc as plsc

N = 16384
D = 64
DH = D // 2
NT = 16
CHUNK = 512
IDXW = 128
NSUB = CHUNK // IDXW
SUP = 8
SROWS = SUP * CHUNK // IDXW
SCH = SUP * CHUNK
RBITS = 14


def _sc_body(nsup, inp_hbm, pk_hbm, sv_hbm, dflat_hbm, out_hbm,
             pkbuf, vbuf, r2d, gat, acc, sem_p, sem_g, sem_s):
    c = lax.axis_index("c")
    s = lax.axis_index("s")
    coff = c * N
    tile_base = s * (nsup * SCH)

    def _zb(i, _):
        gat[0, i, pl.ds(0, 16)] = jnp.zeros((16,), jnp.float32)
        gat[0, i, pl.ds(16, 16)] = jnp.zeros((16,), jnp.float32)
        return _
    lax.fori_loop(0, CHUNK, _zb, None)
    arows = N // NT
    pltpu.sync_copy(gat.at[0], acc.at[pl.ds(s * arows, CHUNK)])
    pltpu.sync_copy(gat.at[0], acc.at[pl.ds(s * arows + CHUNK, CHUNK)])
    plsc.subcore_barrier()

    def meta_copies(u, b):
        base = tile_base + u * SCH
        return [pltpu.make_async_copy(pk_hbm.at[pl.ds(base, SCH)],
                                      pkbuf.at[b], sem_p.at[b]),
                pltpu.make_async_copy(sv_hbm.at[pl.ds(base, SCH)],
                                      vbuf.at[b], sem_p.at[b])]

    def _sup(u, b):
        for cp in meta_copies(u, b):
            cp.wait()
        un = jnp.minimum(u + 1, nsup - 1)
        for cp in meta_copies(un, 1 - b):
            cp.start()

        pend_g, pend_s = {}, {}

        def fire_gather(k):
            g = k % 2

            def _unp(i, _):
                r, l = k * NSUB + i // 8, (i % 8) * 16
                t = pkbuf[b, pl.ds(k * CHUNK + i * 16, 16)]
                r2d[b, r, pl.ds(l, 16)] = t >> RBITS
                pkbuf[b, pl.ds(k * CHUNK + i * 16, 16)] = (t & (N - 1)) + coff
                return _
            lax.fori_loop(0, CHUNK // 16, _unp, None)

            pend_g[k] = [
                pltpu.async_copy(
                    dflat_hbm.at[pkbuf.at[b, pl.ds(k * CHUNK + j * IDXW, IDXW)]],
                    gat.at[g, pl.ds(j * IDXW, IDXW)], sem_g.at[g])
                for j in range(NSUB)]

        def scale_scatter(k):
            g = k % 2
            for cp in pend_g.pop(k):
                cp.wait()

            def _mul(i, _):
                vv = vbuf[b, pl.ds(k * CHUNK + i * 16, 16)]
                for t in range(16):
                    q = i * 16 + t
                    gat[g, q, pl.ds(0, 16)] = gat[g, q, pl.ds(0, 16)] * vv[t]
                    gat[g, q, pl.ds(16, 16)] = gat[g, q, pl.ds(16, 16)] * vv[t]
                return _
            lax.fori_loop(0, CHUNK // 16, _mul, None)

            pend_s[k] = [
                pltpu.async_copy(gat.at[g, pl.ds(j * IDXW, IDXW)],
                                 acc.at[r2d.at[b, k * NSUB + j]],
                                 sem_s.at[g], add=True)
                for j in range(NSUB)]

        for k in range(SUP):
            if k >= 2:
                for cp in pend_s.pop(k - 2):
                    cp.wait()
            fire_gather(k)
            if k >= 1:
                scale_scatter(k - 1)
        scale_scatter(SUP - 1)
        for kk in (SUP - 2, SUP - 1):
            for cp in pend_s.pop(kk):
                cp.wait()

    for cp in meta_copies(0, 0):
        cp.start()

    def _pair(u2, _):
        _sup(2 * u2, 0)
        _sup(2 * u2 + 1, 1)
        return _
    lax.fori_loop(0, nsup // 2, _pair, None)
    if nsup % 2:
        _sup(nsup - 1, 0)
    for cp in meta_copies(nsup - 1, nsup % 2):
        cp.wait()

    plsc.subcore_barrier()

    for half in range(2):
        r0 = s * arows + half * CHUNK
        pltpu.sync_copy(inp_hbm.at[pl.ds(r0, CHUNK), pl.ds(c * DH, DH)],
                        gat.at[0])
        pltpu.sync_copy(acc.at[pl.ds(r0, CHUNK)], gat.at[1])

        def _add(i, _):
            gat[0, i, pl.ds(0, 16)] = gat[0, i, pl.ds(0, 16)] + gat[1, i, pl.ds(0, 16)]
            gat[0, i, pl.ds(16, 16)] = gat[0, i, pl.ds(16, 16)] + gat[1, i, pl.ds(16, 16)]
            return _
        lax.fori_loop(0, CHUNK, _add, None)
        pltpu.sync_copy(gat.at[0],
                        out_hbm.at[pl.ds(r0, CHUNK), pl.ds(c * DH, DH)])


def kernel(input_mat, sparse_indices, sparse_values, dense):
    nnz = sparse_values.shape[0]
    quantum = NT * SCH
    nnz_pad = ((nnz + quantum - 1) // quantum) * quantum
    nsup = nnz_pad // quantum
    pad = nnz_pad - nnz
    ar = jnp.arange(pad, dtype=jnp.int32)
    rows_p = jnp.concatenate([sparse_indices[0], (ar * 97) % N])
    cols_p = jnp.concatenate([sparse_indices[1], (ar * 89) % N])
    vals_p = jnp.pad(sparse_values, (0, pad))
    packed = (rows_p << RBITS) | cols_p
    dflat = jnp.concatenate([dense[:, :DH], dense[:, DH:]], axis=0)

    mesh = plsc.VectorSubcoreMesh(core_axis_name="c", subcore_axis_name="s")
    body = functools.partial(_sc_body, nsup)
    return pl.kernel(
        body,
        out_type=jax.ShapeDtypeStruct((N, D), jnp.float32),
        mesh=mesh,
        compiler_params=pltpu.CompilerParams(use_tc_tiling_on_sc=False,
                                             needs_layout_passes=False),
        scratch_types=[
            pltpu.VMEM((2, SCH), jnp.int32),
            pltpu.VMEM((2, SCH), jnp.float32),
            pltpu.VMEM((2, SROWS, IDXW), jnp.int32),
            pltpu.VMEM((2, CHUNK, DH), jnp.float32),
            pltpu.VMEM_SHARED((N, DH), jnp.float32),
            pltpu.SemaphoreType.DMA((2,)),
            pltpu.SemaphoreType.DMA((2,)),
            pltpu.SemaphoreType.DMA((2,)),
        ],
    )(input_mat, packed, vals_p, dflat)


# speedup vs baseline: 1.4015x; 1.0304x over previous
import functools

import jax
import jax.numpy as jnp
from jax import lax
from jax.experimental import pallas as pl
from jax.experimental.pallas import tpu as pltpu
from jax.experimental.pallas import tpu_sc as plsc

N = 16384
D = 64
DH = D // 2
NT = 16
CHUNK = 512
IDXW = 128
NSUB = CHUNK // IDXW
SUP = 8
SROWS = SUP * CHUNK // IDXW
SCH = SUP * CHUNK
RBITS = 14


def _sc_body(nsup, inp_hbm, pk_hbm, sv_hbm, tpk_hbm, tsv_hbm, dflat_hbm,
             out_hbm, pkbuf, vbuf, r2d, gat, acc, sem_p, sem_g, sem_s):
    c = lax.axis_index("c")
    s = lax.axis_index("s")
    coff = c * N
    tile_base = s * (nsup * SCH)

    def _zb(i, _):
        gat[0, i, pl.ds(0, 16)] = jnp.zeros((16,), jnp.float32)
        gat[0, i, pl.ds(16, 16)] = jnp.zeros((16,), jnp.float32)
        return _
    lax.fori_loop(0, CHUNK, _zb, None)
    arows = N // NT
    pltpu.sync_copy(gat.at[0], acc.at[pl.ds(s * arows, CHUNK)])
    pltpu.sync_copy(gat.at[0], acc.at[pl.ds(s * arows + CHUNK, CHUNK)])
    plsc.subcore_barrier()

    def meta_copies(u, b):
        base = tile_base + u * SCH
        return [pltpu.make_async_copy(pk_hbm.at[pl.ds(base, SCH)],
                                      pkbuf.at[b], sem_p.at[b]),
                pltpu.make_async_copy(sv_hbm.at[pl.ds(base, SCH)],
                                      vbuf.at[b], sem_p.at[b])]

    def _sup(u, b, prefetch=True):
        if prefetch:
            for cp in meta_copies(u, b):
                cp.wait()
            un = jnp.minimum(u + 1, nsup - 1)
            for cp in meta_copies(un, 1 - b):
                cp.start()

        pend_g, pend_s = {}, {}

        def fire_gather(k):
            g = k % 2

            def _unp(i, _):
                r, l = k * NSUB + i // 8, (i % 8) * 16
                t = pkbuf[b, pl.ds(k * CHUNK + i * 16, 16)]
                r2d[b, r, pl.ds(l, 16)] = t >> RBITS
                pkbuf[b, pl.ds(k * CHUNK + i * 16, 16)] = (t & (N - 1)) + coff
                return _
            lax.fori_loop(0, CHUNK // 16, _unp, None)

            pend_g[k] = [
                pltpu.async_copy(
                    dflat_hbm.at[pkbuf.at[b, pl.ds(k * CHUNK + j * IDXW, IDXW)]],
                    gat.at[g, pl.ds(j * IDXW, IDXW)], sem_g.at[g])
                for j in range(NSUB)]

        def scale_scatter(k):
            g = k % 2
            for cp in pend_g.pop(k):
                cp.wait()

            def _mul(i, _):
                vv = vbuf[b, pl.ds(k * CHUNK + i * 16, 16)]
                for t in range(16):
                    q = i * 16 + t
                    gat[g, q, pl.ds(0, 16)] = gat[g, q, pl.ds(0, 16)] * vv[t]
                    gat[g, q, pl.ds(16, 16)] = gat[g, q, pl.ds(16, 16)] * vv[t]
                return _
            lax.fori_loop(0, CHUNK // 16, _mul, None)

            pend_s[k] = [
                pltpu.async_copy(gat.at[g, pl.ds(j * IDXW, IDXW)],
                                 acc.at[r2d.at[b, k * NSUB + j]],
                                 sem_s.at[g], add=True)
                for j in range(NSUB)]

        for k in range(SUP):
            if k >= 2:
                for cp in pend_s.pop(k - 2):
                    cp.wait()
            fire_gather(k)
            if k >= 1:
                scale_scatter(k - 1)
        scale_scatter(SUP - 1)
        for kk in (SUP - 2, SUP - 1):
            for cp in pend_s.pop(kk):
                cp.wait()

    if nsup > 0:
        for cp in meta_copies(0, 0):
            cp.start()

        def _pair(u2, _):
            _sup(2 * u2, 0)
            _sup(2 * u2 + 1, 1)
            return _
        lax.fori_loop(0, nsup // 2, _pair, None)
        if nsup % 2:
            _sup(nsup - 1, 0)
        for cp in meta_copies(nsup - 1, nsup % 2):
            cp.wait()

    pltpu.sync_copy(tpk_hbm.at[pl.ds(s * SCH, SCH)], pkbuf.at[0])
    pltpu.sync_copy(tsv_hbm.at[pl.ds(s * SCH, SCH)], vbuf.at[0])
    _sup(0, 0, prefetch=False)

    plsc.subcore_barrier()

    for half in range(2):
        r0 = s * arows + half * CHUNK
        pltpu.sync_copy(inp_hbm.at[pl.ds(r0, CHUNK), pl.ds(c * DH, DH)],
                        gat.at[0])
        pltpu.sync_copy(acc.at[pl.ds(r0, CHUNK)], gat.at[1])

        def _add(i, _):
            gat[0, i, pl.ds(0, 16)] = gat[0, i, pl.ds(0, 16)] + gat[1, i, pl.ds(0, 16)]
            gat[0, i, pl.ds(16, 16)] = gat[0, i, pl.ds(16, 16)] + gat[1, i, pl.ds(16, 16)]
            return _
        lax.fori_loop(0, CHUNK, _add, None)
        pltpu.sync_copy(gat.at[0],
                        out_hbm.at[pl.ds(r0, CHUNK), pl.ds(c * DH, DH)])


def kernel(input_mat, sparse_indices, sparse_values, dense):
    nnz = sparse_values.shape[0]
    quantum = NT * SCH
    nsup = nnz // quantum
    bulk = nsup * quantum
    pad = quantum - (nnz - bulk)
    packed = (sparse_indices[0] << RBITS) | sparse_indices[1]
    ar = jnp.arange(pad, dtype=jnp.int32)
    tpk = jnp.concatenate(
        [packed[bulk:], (((ar * 97) % N) << RBITS) | ((ar * 89) % N)])
    tsv = jnp.pad(sparse_values[bulk:], (0, pad))
    dflat = jnp.concatenate([dense[:, :DH], dense[:, DH:]], axis=0)

    mesh = plsc.VectorSubcoreMesh(core_axis_name="c", subcore_axis_name="s")
    body = functools.partial(_sc_body, nsup)
    return pl.kernel(
        body,
        out_type=jax.ShapeDtypeStruct((N, D), jnp.float32),
        mesh=mesh,
        compiler_params=pltpu.CompilerParams(use_tc_tiling_on_sc=False,
                                             needs_layout_passes=False),
        scratch_types=[
            pltpu.VMEM((2, SCH), jnp.int32),
            pltpu.VMEM((2, SCH), jnp.float32),
            pltpu.VMEM((2, SROWS, IDXW), jnp.int32),
            pltpu.VMEM((2, CHUNK, DH), jnp.float32),
            pltpu.VMEM_SHARED((N, DH), jnp.float32),
            pltpu.SemaphoreType.DMA((2,)),
            pltpu.SemaphoreType.DMA((2,)),
            pltpu.SemaphoreType.DMA((2,)),
        ],
    )(input_mat, packed, sparse_values, tpk, tsv, dflat)
